# Initial kernel scaffold; baseline (speedup 1.0000x reference)
import jax
import jax.numpy as jnp
from jax.experimental import pallas as pl


def kernel(input_ids, token_type_ids, token_w, pos_w, type_w, ln_w, ln_b):
    raise NotImplementedError("write your pallas kernel here")


# speedup vs baseline: 4.7109x; 4.7109x over previous
import functools

import jax
import jax.numpy as jnp
from jax import lax
from jax.experimental import pallas as pl
from jax.experimental.pallas import tpu as pltpu
from jax.experimental.pallas import tpu_sc as plsc

VOCAB = 100000
HIDDEN = 128
MAX_POS = 512
B, T = 1024, 200
N_TOK = B * T

NUM_CORES = 2
NUM_SUBCORES = 16
NUM_WORKERS = NUM_CORES * NUM_SUBCORES
CHUNK = 128
PER_WORKER = N_TOK // NUM_WORKERS
CHUNKS_PER_WORKER = PER_WORKER // CHUNK


def _sc_gather_body(ids_hbm, table_hbm, out_hbm, idx_v, rows_v, sem):
    wid = lax.axis_index("s") * NUM_CORES + lax.axis_index("c")
    base = wid * PER_WORKER

    def body(i, carry):
        start = base + i * CHUNK
        pltpu.sync_copy(ids_hbm.at[pl.ds(start, CHUNK)], idx_v)
        pltpu.async_copy(table_hbm.at[idx_v], rows_v, sem).wait()
        pltpu.sync_copy(rows_v, out_hbm.at[pl.ds(start, CHUNK)])
        return carry

    lax.fori_loop(0, CHUNKS_PER_WORKER, body, 0)


def _sc_gather(flat_ids, token_w):
    mesh = plsc.VectorSubcoreMesh(core_axis_name="c", subcore_axis_name="s")
    k = functools.partial(
        pl.kernel,
        mesh=mesh,
        out_type=jax.ShapeDtypeStruct((N_TOK, HIDDEN), jnp.float32),
        scratch_types=[
            pltpu.VMEM((CHUNK,), jnp.int32),
            pltpu.VMEM((CHUNK, HIDDEN), jnp.float32),
            pltpu.SemaphoreType.DMA,
        ],
    )(_sc_gather_body)
    return k(flat_ids, token_w)


def _tc_ln_body(g_ref, ttf_ref, pos_ref, type_ref, lnw_ref, lnb_ref, o_ref):
    g = g_ref[...]
    ttf = ttf_ref[...]
    pos = pos_ref[...]
    t0 = type_ref[0, :][None, None, :]
    t1 = type_ref[1, :][None, None, :]
    te = t0 + ttf * (t1 - t0)
    x = g + pos[None] + te
    mean = jnp.mean(x, axis=-1, keepdims=True)
    xc = x - mean
    var = jnp.mean(xc * xc, axis=-1, keepdims=True)
    y = xc * lax.rsqrt(var + 1e-5)
    o_ref[...] = y * lnw_ref[...] + lnb_ref[...]


def _tc_ln(gathered, token_type_f, pos_w, type_w, ln_w, ln_b):
    BB = 8
    grid = (B // BB,)
    return pl.pallas_call(
        _tc_ln_body,
        grid=grid,
        in_specs=[
            pl.BlockSpec((BB, T, HIDDEN), lambda i: (i, 0, 0)),
            pl.BlockSpec((BB, T, 1), lambda i: (i, 0, 0)),
            pl.BlockSpec((T, HIDDEN), lambda i: (0, 0)),
            pl.BlockSpec((2, HIDDEN), lambda i: (0, 0)),
            pl.BlockSpec((HIDDEN,), lambda i: (0,)),
            pl.BlockSpec((HIDDEN,), lambda i: (0,)),
        ],
        out_specs=pl.BlockSpec((BB, T, HIDDEN), lambda i: (i, 0, 0)),
        out_shape=jax.ShapeDtypeStruct((B, T, HIDDEN), jnp.float32),
    )(gathered, token_type_f, pos_w, type_w, ln_w, ln_b)


def kernel(input_ids, token_type_ids, token_w, pos_w, type_w, ln_w, ln_b):
    flat_ids = input_ids.reshape(-1)
    gathered = _sc_gather(flat_ids, token_w)
    gathered = gathered.reshape(B, T, HIDDEN)
    ttf = token_type_ids.astype(jnp.float32).reshape(B, T, 1)
    return _tc_ln(gathered, ttf, pos_w, type_w, ln_w, ln_b)
